# baseline (speedup 1.0000x reference)
import functools

import jax
import jax.numpy as jnp
from jax import lax
from jax.experimental import pallas as pl
from jax.experimental.pallas import tpu as pltpu
from jax.experimental.pallas import tpu_sc as plsc

_C = 1000
_B = 1024
_ROWS = 40
_NTASK = (50 * _C) // _ROWS
_NW = 32
_KPW = 20


def _scatter_row(chunk, row, c0, value):
    lane = lax.iota(jnp.int32, 16)
    vals = jnp.full((16,), value, jnp.float32)

    def body(g, _):
        tv = row[pl.ds(g * 16, 16)]
        rel = tv - c0
        mask = (rel >= 0) & (rel < _ROWS)
        idx_b = g * 16 + lane
        plsc.store_scatter(chunk, [rel, idx_b], vals, mask=mask)
        return 0

    lax.fori_loop(0, _B // 16, body, 0, unroll=4)


def _task_ids(w, kk, b):
    task = jnp.minimum(w + (kk * 2 + b) * _NW, _NTASK - 1)
    s = task // (_C // _ROWS)
    c0 = (task % (_C // _ROWS)) * _ROWS
    return s, c0


def _sc_onehot(tt_hbm, out_hbm, chunks, row, sems):
    w = lax.axis_index("s") * 2 + lax.axis_index("c")

    zeros16 = jnp.zeros((16,), jnp.float32)

    def zbody(i, _):
        c = i // (_B // 16)
        g = i % (_B // 16)
        chunks[0, c, pl.ds(g * 16, 16)] = zeros16
        chunks[1, c, pl.ds(g * 16, 16)] = zeros16
        return 0

    lax.fori_loop(0, _ROWS * (_B // 16), zbody, 0, unroll=8)

    def pair(kk, _):
        for b in (0, 1):
            chunk = chunks.at[b]
            s, c0 = _task_ids(w, kk, b)

            @pl.when(kk > 0)
            def _reclaim():
                pltpu.make_async_copy(
                    chunk, out_hbm.at[0, pl.ds(0, _ROWS)], sems.at[b]
                ).wait()
                ps, pc0 = _task_ids(w, kk - 1, b)
                pltpu.sync_copy(tt_hbm.at[ps], row)
                _scatter_row(chunk, row, pc0, 0.0)

            pltpu.sync_copy(tt_hbm.at[s], row)
            _scatter_row(chunk, row, c0, 1.0)
            pltpu.make_async_copy(
                chunk, out_hbm.at[s, pl.ds(c0, _ROWS)], sems.at[b]
            ).start()
        return 0

    lax.fori_loop(0, _KPW, pair, 0)

    for b in (0, 1):
        pltpu.make_async_copy(
            chunks.at[b], out_hbm.at[0, pl.ds(0, _ROWS)], sems.at[b]
        ).wait()


def kernel(t) -> jnp.ndarray:
    B, S = t.shape
    tt = t.astype(jnp.int32).T
    mesh = plsc.VectorSubcoreMesh(core_axis_name="c", subcore_axis_name="s")
    run = functools.partial(
        pl.kernel,
        mesh=mesh,
        compiler_params=pltpu.CompilerParams(needs_layout_passes=False),
        out_type=jax.ShapeDtypeStruct((S, _C, B), jnp.float32),
        scratch_types=[
            pltpu.VMEM((2, _ROWS, B), jnp.float32),
            pltpu.VMEM((B,), jnp.int32),
            pltpu.SemaphoreType.DMA((2,)),
        ],
    )(_sc_onehot)
    out_t = run(tt)
    return jnp.transpose(out_t, (2, 1, 0))


# speedup vs baseline: 2.5504x; 2.5504x over previous
import jax
import jax.numpy as jnp
from jax.experimental import pallas as pl
from jax.experimental.pallas import tpu as pltpu

_N_CLASSES = 1000
_NBUF = 8


def _onehot_block(t_ref, out_ref, slabs, sems):
    i = pl.program_id(0)
    n = pl.num_programs(0)
    h = 504
    C = out_ref.shape[1]
    slot = jax.lax.rem(i, _NBUF)

    def _copies(step, slab_slot, sem_slot):
        return (
            pltpu.make_async_copy(
                slabs.at[slab_slot, :, pl.ds(0, h)],
                out_ref.at[pl.ds(step, 1), pl.ds(0, h)],
                sems.at[sem_slot, 0],
            ),
            pltpu.make_async_copy(
                slabs.at[slab_slot, :, pl.ds(h, C - h)],
                out_ref.at[pl.ds(step, 1), pl.ds(h, C - h)],
                sems.at[sem_slot, 1],
            ),
        )

    @pl.when(i >= _NBUF)
    def _wait_prev():
        for cp in _copies(i, slot, slot):
            cp.wait()

    t_row = t_ref[...]
    c = jax.lax.broadcasted_iota(jnp.int32, slabs.shape[1:], 1)
    slabs[slot] = (c == t_row).astype(jnp.float32)

    for cp in _copies(i, slot, slot):
        cp.start()

    @pl.when(i == n - 1)
    def _drain():
        for k in range(_NBUF):
            for cp in _copies(0, k, k):
                cp.wait()


def kernel(t) -> jnp.ndarray:
    B, S = t.shape
    C = _N_CLASSES
    tt = t.astype(jnp.int32).T.reshape(S, 1, B)
    out_t = pl.pallas_call(
        _onehot_block,
        grid=(S,),
        in_specs=[pl.BlockSpec((1, 1, B), lambda i: (i, 0, 0))],
        out_specs=pl.BlockSpec(memory_space=pl.ANY),
        out_shape=jax.ShapeDtypeStruct((S, C, B), jnp.float32),
        scratch_shapes=[
            pltpu.VMEM((_NBUF, 1, C, B), jnp.float32),
            pltpu.SemaphoreType.DMA((_NBUF, 2)),
        ],
    )(tt)
    return jnp.transpose(out_t, (2, 1, 0))


# speedup vs baseline: 2.5981x; 1.0187x over previous
import jax
import jax.numpy as jnp
from jax.experimental import pallas as pl
from jax.experimental.pallas import tpu as pltpu

_N_CLASSES = 1000
_NBUF = 8


def _onehot_block(t_ref, out_ref, slabs, sems):
    i = pl.program_id(0)
    n = pl.num_programs(0)
    h = 504
    C = out_ref.shape[1]
    slot = jax.lax.rem(i, _NBUF)

    def _copies(step, slab_slot, sem_slot):
        return (
            pltpu.make_async_copy(
                slabs.at[slab_slot, :, pl.ds(0, h)],
                out_ref.at[pl.ds(step, 1), pl.ds(0, h)],
                sems.at[sem_slot, 0],
            ),
            pltpu.make_async_copy(
                slabs.at[slab_slot, :, pl.ds(h, C - h)],
                out_ref.at[pl.ds(step, 1), pl.ds(h, C - h)],
                sems.at[sem_slot, 1],
            ),
        )

    @pl.when(i >= _NBUF)
    def _wait_prev():
        for cp in _copies(i, slot, slot):
            cp.wait()

    t_row = t_ref[pl.ds(i, 1), :]
    c = jax.lax.broadcasted_iota(jnp.int32, slabs.shape[1:], 1)
    slabs[slot] = (c == t_row[:, None, :]).astype(jnp.float32)

    for cp in _copies(i, slot, slot):
        cp.start()

    @pl.when(i == n - 1)
    def _drain():
        for k in range(_NBUF):
            for cp in _copies(0, k, k):
                cp.wait()


def kernel(t) -> jnp.ndarray:
    B, S = t.shape
    C = _N_CLASSES
    tt = t.astype(jnp.int32).T
    out_t = pl.pallas_call(
        _onehot_block,
        grid=(S,),
        in_specs=[pl.BlockSpec(memory_space=pltpu.VMEM)],
        out_specs=pl.BlockSpec(memory_space=pl.ANY),
        out_shape=jax.ShapeDtypeStruct((S, C, B), jnp.float32),
        scratch_shapes=[
            pltpu.VMEM((_NBUF, 1, C, B), jnp.float32),
            pltpu.SemaphoreType.DMA((_NBUF, 2)),
        ],
    )(tt)
    return jnp.transpose(out_t, (2, 1, 0))
